# Initial kernel scaffold; baseline (speedup 1.0000x reference)
import functools

import jax
import jax.numpy as jnp
from jax import lax
from jax.experimental import pallas as pl
from jax.experimental.pallas import tpu as pltpu
from jax.experimental.pallas import tpu_sc as plsc

NUM_CORES = 2
NUM_SUBCORES = 16
NUM_WORKERS = NUM_CORES * NUM_SUBCORES

B_TOTAL = 16384 * 26
EMB_DIM = 32
B_PER_W = B_TOTAL // NUM_WORKERS
CHUNK = 1664
NCHUNK = B_PER_W // CHUNK


def _make_gather():
  mesh = plsc.VectorSubcoreMesh(core_axis_name="c", subcore_axis_name="s")

  @functools.partial(
      pl.kernel,
      out_type=jax.ShapeDtypeStruct((B_TOTAL, EMB_DIM), jnp.float32),
      mesh=mesh,
      scratch_types=[
          pltpu.VMEM((CHUNK,), jnp.int32),
          pltpu.VMEM((CHUNK, EMB_DIM), jnp.float32),
          pltpu.SemaphoreType.DMA,
      ],
  )
  def gather_kernel(idx_hbm, table_hbm, out_hbm, idx_v, rows_v, sem):
    wid = lax.axis_index("s") * NUM_CORES + lax.axis_index("c")
    base = wid * B_PER_W

    def chunk_body(i, carry):
      off = pl.multiple_of(base + i * CHUNK, 8)
      pltpu.sync_copy(idx_hbm.at[pl.ds(off, CHUNK)], idx_v)
      pltpu.async_copy(table_hbm.at[idx_v], rows_v, sem).wait()
      pltpu.sync_copy(rows_v, out_hbm.at[pl.ds(off, CHUNK)])
      return carry

    lax.fori_loop(0, NCHUNK, chunk_body, 0)

  return gather_kernel


_gather = _make_gather()


def kernel(indices, weight):
  idx_flat = indices.reshape(-1).astype(jnp.int32)
  out = _gather(idx_flat, weight)
  return out.reshape(indices.shape[0], indices.shape[1], weight.shape[1])


# speedup vs baseline: 1.5620x; 1.5620x over previous
import functools

import jax
import jax.numpy as jnp
from jax import lax
from jax.experimental import pallas as pl
from jax.experimental.pallas import tpu as pltpu
from jax.experimental.pallas import tpu_sc as plsc

NUM_CORES = 2
NUM_SUBCORES = 16
NUM_WORKERS = NUM_CORES * NUM_SUBCORES

B_TOTAL = 16384 * 26
EMB_DIM = 32
B_PER_W = B_TOTAL // NUM_WORKERS
CHUNK = 1664
NCHUNK = B_PER_W // CHUNK


def _make_gather():
  mesh = plsc.VectorSubcoreMesh(core_axis_name="c", subcore_axis_name="s")

  @functools.partial(
      pl.kernel,
      out_type=jax.ShapeDtypeStruct((B_TOTAL, EMB_DIM), jnp.float32),
      mesh=mesh,
      compiler_params=pltpu.CompilerParams(use_tc_tiling_on_sc=False),
      scratch_types=[
          pltpu.VMEM((CHUNK,), jnp.int32),
          pltpu.VMEM((CHUNK, EMB_DIM), jnp.float32),
          pltpu.SemaphoreType.DMA,
      ],
  )
  def gather_kernel(idx_hbm, table_hbm, out_hbm, idx_v, rows_v, sem):
    wid = lax.axis_index("s") * NUM_CORES + lax.axis_index("c")
    base = wid * B_PER_W

    def chunk_body(i, carry):
      off = pl.multiple_of(base + i * CHUNK, 8)
      pltpu.sync_copy(idx_hbm.at[pl.ds(off, CHUNK)], idx_v)
      pltpu.async_copy(table_hbm.at[idx_v], rows_v, sem).wait()
      pltpu.sync_copy(rows_v, out_hbm.at[pl.ds(off, CHUNK)])
      return carry

    lax.fori_loop(0, NCHUNK, chunk_body, 0)

  return gather_kernel


_gather = _make_gather()


def kernel(indices, weight):
  idx_flat = indices.reshape(-1).astype(jnp.int32)
  out = _gather(idx_flat, weight)
  return out.reshape(indices.shape[0], indices.shape[1], weight.shape[1])


# speedup vs baseline: 1.5756x; 1.0087x over previous
import functools

import jax
import jax.numpy as jnp
from jax import lax
from jax.experimental import pallas as pl
from jax.experimental.pallas import tpu as pltpu
from jax.experimental.pallas import tpu_sc as plsc

NUM_CORES = 2
NUM_SUBCORES = 16
NUM_WORKERS = NUM_CORES * NUM_SUBCORES

B_TOTAL = 16384 * 26
EMB_DIM = 32
B_PER_W = B_TOTAL // NUM_WORKERS
CHUNK = 832
NCHUNK = B_PER_W // CHUNK
NBUF = 4


def _make_gather():
  mesh = plsc.VectorSubcoreMesh(core_axis_name="c", subcore_axis_name="s")

  @functools.partial(
      pl.kernel,
      out_type=jax.ShapeDtypeStruct((B_TOTAL, EMB_DIM), jnp.float32),
      mesh=mesh,
      compiler_params=pltpu.CompilerParams(use_tc_tiling_on_sc=False),
      scratch_types=[
          pltpu.VMEM((B_PER_W,), jnp.int32),
          pltpu.VMEM((NBUF, CHUNK, EMB_DIM), jnp.float32),
      ] + [pltpu.SemaphoreType.DMA] * (2 * NBUF),
  )
  def gather_kernel(idx_hbm, table_hbm, out_hbm, idx_v, rows_v, *sems):
    gsem, ssem = sems[:NBUF], sems[NBUF:]
    wid = lax.axis_index("s") * NUM_CORES + lax.axis_index("c")
    base = pl.multiple_of(wid * B_PER_W, 8)
    pltpu.sync_copy(idx_hbm.at[pl.ds(base, B_PER_W)], idx_v)

    def start_gather(i, b):
      return pltpu.async_copy(
          table_hbm.at[idx_v.at[pl.ds(i * CHUNK, CHUNK)]],
          rows_v.at[b], gsem[b])

    def start_store(i, b):
      return pltpu.async_copy(
          rows_v.at[b],
          out_hbm.at[pl.ds(pl.multiple_of(base + i * CHUNK, 8), CHUNK)],
          ssem[b])

    gh = [start_gather(b, b) for b in range(NBUF)]
    sh = [None] * NBUF
    for i in range(NCHUNK):
      b = i % NBUF
      gh[b].wait()
      sh[b] = start_store(i, b)
      j = i + NBUF
      if j < NCHUNK:
        sh[b].wait()
        gh[b] = start_gather(j, b)
    for i in range(NCHUNK - NBUF, NCHUNK):
      sh[i % NBUF].wait()

  return gather_kernel


_gather = _make_gather()


def kernel(indices, weight):
  idx_flat = indices.reshape(-1).astype(jnp.int32)
  out = _gather(idx_flat, weight)
  return out.reshape(indices.shape[0], indices.shape[1], weight.shape[1])
